# Initial kernel scaffold; baseline (speedup 1.0000x reference)
import jax
import jax.numpy as jnp
from jax.experimental import pallas as pl


def kernel(inputs, We1, be1, We2, be2, Wd1, bd1, Wd2, bd2, cb):
    raise NotImplementedError("write your pallas kernel here")


# speedup vs baseline: 1.8184x; 1.8184x over previous
import functools

import jax
import jax.numpy as jnp
from jax.experimental import pallas as pl

DIST_D = 128
BIN_SIZE = 500
K = 8
NBINS = 16


def _elu(x):
    return jnp.where(x > 0, x, jnp.exp(jnp.minimum(x, 0.0)) - 1.0)


def _stage1_body(x_ref, We1_ref, be1_ref, We2_ref, be2_ref, cb8_ref,
                 Wd1a_ref, Wd1b_ref, bd1_ref,
                 emb_ref, bin_ref, p1_ref, p2_ref):
    x = x_ref[...]
    h = _elu(x @ We1_ref[...] + be1_ref[...])
    emb = h @ We2_ref[...] + be2_ref[...]
    emb_ref[...] = emb
    mul = emb @ cb8_ref[...]
    cmul = jnp.concatenate([mul, -mul], axis=1)
    mx = jnp.max(cmul, axis=1, keepdims=True)
    iota = jax.lax.broadcasted_iota(jnp.int32, cmul.shape, 1)
    bin_ref[0, 0, :] = jnp.min(jnp.where(cmul == mx, iota, NBINS), axis=1)
    p1_ref[...] = x @ Wd1a_ref[...] + bd1_ref[...]
    p2_ref[...] = x @ Wd1b_ref[...]


def _stage3_body(parts_ref, vals_ref, idx_ref):
    pt = parts_ref[0, 0]
    s = jax.lax.dot_general(pt, pt, (((1,), (1,)), ((), ())))
    dm = jax.nn.sigmoid(s)
    iota = jax.lax.broadcasted_iota(jnp.int32, dm.shape, 1)
    for j in range(K):
        mx = jnp.max(dm, axis=1, keepdims=True)
        am = jnp.min(jnp.where(dm == mx, iota, BIN_SIZE), axis=1)
        vals_ref[0, 0, j, :] = mx[:, 0]
        idx_ref[0, 0, j, :] = am
        dm = jnp.where(iota == am[:, None], -jnp.inf, dm)


def _stage5_body(g_ref, p1_ref, vals_ref, w512_ref, Wd2_ref, bd2_ref, out_ref):
    rows = p1_ref.shape[0]
    g = g_ref[...].reshape(rows, K, DIST_D)
    z = g + p1_ref[...][:, None, :]
    z = z.reshape(rows * K, DIST_D)
    z = z + vals_ref[0, 0, :][:, None] * w512_ref[...]
    h = _elu(z)
    y = h @ Wd2_ref[...] + bd2_ref[0, 0]
    out_ref[0, 0, :] = jax.nn.sigmoid(y)[:, 0]


def kernel(inputs, We1, be1, We2, be2, Wd1, bd1, Wd2, bd2, cb):
    B, N, F = inputs.shape
    n_bins = N // BIN_SIZE
    BN = B * N
    BLK = 1000
    nblk = BN // BLK

    cb8 = cb[:, : n_bins // 2]
    Wd1a = Wd1[:F]
    Wd1b = Wd1[F:2 * F]
    w512 = Wd1[2 * F:2 * F + 1]

    x2 = inputs.reshape(BN, F)

    full = lambda shape: pl.BlockSpec(shape, lambda i: (0,) * len(shape))
    emb2, bins3, p1_2, p2_2 = pl.pallas_call(
        _stage1_body,
        grid=(nblk,),
        in_specs=[
            pl.BlockSpec((BLK, F), lambda i: (i, 0)),
            full((F, 128)), full((128,)), full((128, DIST_D)), full((DIST_D,)),
            full((DIST_D, n_bins // 2)),
            full((F, 128)), full((F, 128)), full((128,)),
        ],
        out_specs=[
            pl.BlockSpec((BLK, DIST_D), lambda i: (i, 0)),
            pl.BlockSpec((1, 1, BLK), lambda i: (i, 0, 0)),
            pl.BlockSpec((BLK, 128), lambda i: (i, 0)),
            pl.BlockSpec((BLK, 128), lambda i: (i, 0)),
        ],
        out_shape=[
            jax.ShapeDtypeStruct((BN, DIST_D), jnp.float32),
            jax.ShapeDtypeStruct((nblk, 1, BLK), jnp.int32),
            jax.ShapeDtypeStruct((BN, 128), jnp.float32),
            jax.ShapeDtypeStruct((BN, 128), jnp.float32),
        ],
    )(x2, We1, be1, We2, be2, cb8, Wd1a, Wd1b, bd1)

    emb = emb2.reshape(B, N, DIST_D)
    bin_idx = bins3.reshape(B, N)
    p1 = p1_2.reshape(B, N, 128)
    p2 = p2_2.reshape(B, N, 128)

    bins_split = jnp.argsort(bin_idx, axis=1).astype(jnp.int32)
    parts = jnp.take_along_axis(emb, bins_split[:, :, None], axis=1)
    parts = parts.reshape(B, n_bins, BIN_SIZE, DIST_D)

    valsT, idxT = pl.pallas_call(
        _stage3_body,
        grid=(B, n_bins),
        in_specs=[pl.BlockSpec((1, 1, BIN_SIZE, DIST_D), lambda b, c: (b, c, 0, 0))],
        out_specs=[
            pl.BlockSpec((1, 1, K, BIN_SIZE), lambda b, c: (b, c, 0, 0)),
            pl.BlockSpec((1, 1, K, BIN_SIZE), lambda b, c: (b, c, 0, 0)),
        ],
        out_shape=[
            jax.ShapeDtypeStruct((B, n_bins, K, BIN_SIZE), jnp.float32),
            jax.ShapeDtypeStruct((B, n_bins, K, BIN_SIZE), jnp.int32),
        ],
    )(parts)

    idx_p = jnp.transpose(idxT, (0, 1, 3, 2)).reshape(B, N, K)
    vals_p = jnp.transpose(valsT, (0, 1, 3, 2)).reshape(B, N, K)
    base = (jnp.arange(N, dtype=jnp.int32) // BIN_SIZE * BIN_SIZE)[None, :, None]
    dst_p = jnp.take_along_axis(bins_split, (idx_p + base).reshape(B, N * K),
                                axis=1).reshape(B, N, K)
    inv = jnp.argsort(bins_split, axis=1).astype(jnp.int32)
    cols_r = jnp.take_along_axis(dst_p.reshape(B, N * K),
                                 (inv[:, :, None] * K +
                                  jnp.arange(K, dtype=jnp.int32)).reshape(B, N * K),
                                 axis=1).reshape(B, N, K)
    vals_r = jnp.take_along_axis(vals_p.reshape(B, N * K),
                                 (inv[:, :, None] * K +
                                  jnp.arange(K, dtype=jnp.int32)).reshape(B, N * K),
                                 axis=1).reshape(B, N, K)
    order = jnp.argsort(cols_r, axis=2)
    cols_s = jnp.take_along_axis(cols_r, order, axis=2)
    vals_s = jnp.take_along_axis(vals_r, order, axis=2)
    g = jnp.take_along_axis(p2, cols_s.reshape(B, N * K)[:, :, None], axis=1)

    E = B * N * K
    EB = BLK * K
    g2 = g.reshape(E, 128)
    vals_flat = vals_s.reshape(nblk, 1, EB)
    edge_vals = pl.pallas_call(
        _stage5_body,
        grid=(nblk,),
        in_specs=[
            pl.BlockSpec((EB, 128), lambda i: (i, 0)),
            pl.BlockSpec((BLK, 128), lambda i: (i, 0)),
            pl.BlockSpec((1, 1, EB), lambda i: (i, 0, 0)),
            full((1, 128)), full((128, 1)),
            pl.BlockSpec((1, 1), lambda i: (0, 0)),
        ],
        out_specs=pl.BlockSpec((1, 1, EB), lambda i: (i, 0, 0)),
        out_shape=jax.ShapeDtypeStruct((nblk, 1, EB), jnp.float32),
    )(g2, p1_2, vals_flat, w512, Wd2, bd2.reshape(1, 1))
    edge_vals = edge_vals.reshape(E)

    batch_col = jnp.repeat(jnp.arange(B, dtype=jnp.int32), N * K)
    row_col = jnp.tile(jnp.repeat(jnp.arange(N, dtype=jnp.int32), K), B)
    indices = jnp.stack([batch_col, row_col, cols_s.reshape(E)], axis=1)
    return indices, edge_vals


# speedup vs baseline: 7.2826x; 4.0049x over previous
import functools

import jax
import jax.numpy as jnp
from jax import lax
from jax.experimental import pallas as pl
from jax.experimental.pallas import tpu as pltpu
from jax.experimental.pallas import tpu_sc as plsc

DIST_D = 128
BIN_SIZE = 500
K = 8
NBINS = 16


def _elu(x):
    return jnp.where(x > 0, x, jnp.exp(jnp.minimum(x, 0.0)) - 1.0)


def _stage1_body(x_ref, We1_ref, be1_ref, We2_ref, be2_ref, cb8_ref,
                 Wd1a_ref, Wd1b_ref, bd1_ref,
                 emb_ref, bin_ref, p1_ref, p2_ref):
    x = x_ref[...]
    h = _elu(x @ We1_ref[...] + be1_ref[...])
    emb = h @ We2_ref[...] + be2_ref[...]
    emb_ref[...] = emb
    mul = emb @ cb8_ref[...]
    cmul = jnp.concatenate([mul, -mul], axis=1)
    mx = jnp.max(cmul, axis=1, keepdims=True)
    iota = jax.lax.broadcasted_iota(jnp.int32, cmul.shape, 1)
    bin_ref[0, 0, :] = jnp.min(jnp.where(cmul == mx, iota, NBINS), axis=1)
    p1_ref[...] = x @ Wd1a_ref[...] + bd1_ref[...]
    p2_ref[...] = x @ Wd1b_ref[...]


def _stage3_body(parts_ref, vals_ref, idx_ref):
    pt = parts_ref[0, 0]
    s = jax.lax.dot_general(pt, pt, (((1,), (1,)), ((), ())))
    dm = jax.nn.sigmoid(s)
    iota = jax.lax.broadcasted_iota(jnp.int32, dm.shape, 1)
    for j in range(K):
        mx = jnp.max(dm, axis=1, keepdims=True)
        am = jnp.min(jnp.where(dm == mx, iota, BIN_SIZE), axis=1)
        vals_ref[0, 0, j, :] = mx[:, 0]
        idx_ref[0, 0, j, :] = am
        dm = jnp.where(iota == am[:, None], -jnp.inf, dm)


def _stage5_body(g_ref, p1_ref, vals_ref, w512_ref, Wd2_ref, bd2_ref, out_ref):
    rows = p1_ref.shape[0]
    g = g_ref[...].reshape(rows, K, DIST_D)
    z = g + p1_ref[...][:, None, :]
    z = z.reshape(rows * K, DIST_D)
    z = z + vals_ref[0, 0, :][:, None] * w512_ref[...]
    h = _elu(z)
    y = h @ Wd2_ref[...] + bd2_ref[0, 0]
    out_ref[0, 0, :] = jax.nn.sigmoid(y)[:, 0]


def _sc4_body(idxT, valsT, bins, p2, edst3,
              cols_out, valb_out, gout,
              idx_v, vals_v, bins_v, cols_v, valso_v, edst_v, g_v, sem):
    wid = lax.axis_index("s") * 2 + lax.axis_index("c")
    b = wid // NBINS
    bn = wid % NBINS
    pw = bn * BIN_SIZE
    pltpu.sync_copy(idxT.at[wid], idx_v)
    pltpu.sync_copy(valsT.at[wid], vals_v)
    pltpu.sync_copy(bins.at[b], bins_v)
    pltpu.sync_copy(edst3.at[wid], edst_v)

    lane = lax.iota(jnp.int32, 16)
    j8 = lane & 7
    r2 = lane >> 3
    hi = r2 * 65536

    def body(t, carry):
        rowsel = 2 * t + r2
        loc = plsc.load_gather(idx_v, [j8 * BIN_SIZE + rowsel])
        val = plsc.load_gather(vals_v, [j8 * BIN_SIZE + rowsel])
        col = plsc.load_gather(bins_v, [pw + loc])
        ksort, vsort = plsc.sort_key_val(col + hi, val)
        cols_v[pl.ds(16 * t, 16)] = b * 8000 + (ksort - hi)
        valso_v[pl.ds(16 * t, 16)] = vsort
        return carry

    lax.fori_loop(0, BIN_SIZE // 2, body, 0)

    pending = []
    gather_done = None
    for c in range(50):
        sl = pl.ds(c * 80, 80)
        pending.append(pltpu.async_copy(cols_v.at[sl], cols_out.at[edst_v.at[c]], sem))
        pending.append(pltpu.async_copy(valso_v.at[sl], valb_out.at[edst_v.at[c]], sem))
        pltpu.async_copy(p2.at[cols_v.at[sl]], g_v, sem).wait()
        pltpu.async_copy(g_v, gout.at[edst_v.at[c]], sem).wait()
    for d in pending:
        d.wait()


def kernel(inputs, We1, be1, We2, be2, Wd1, bd1, Wd2, bd2, cb):
    B, N, F = inputs.shape
    n_bins = N // BIN_SIZE
    BN = B * N
    BLK = 1000
    nblk = BN // BLK

    cb8 = cb[:, : n_bins // 2]
    Wd1a = Wd1[:F]
    Wd1b = Wd1[F:2 * F]
    w512 = Wd1[2 * F:2 * F + 1]

    x2 = inputs.reshape(BN, F)

    full = lambda shape: pl.BlockSpec(shape, lambda i: (0,) * len(shape))
    emb2, bins3, p1_2, p2_2 = pl.pallas_call(
        _stage1_body,
        grid=(nblk,),
        in_specs=[
            pl.BlockSpec((BLK, F), lambda i: (i, 0)),
            full((F, 128)), full((128,)), full((128, DIST_D)), full((DIST_D,)),
            full((DIST_D, n_bins // 2)),
            full((F, 128)), full((F, 128)), full((128,)),
        ],
        out_specs=[
            pl.BlockSpec((BLK, DIST_D), lambda i: (i, 0)),
            pl.BlockSpec((1, 1, BLK), lambda i: (i, 0, 0)),
            pl.BlockSpec((BLK, 128), lambda i: (i, 0)),
            pl.BlockSpec((BLK, 128), lambda i: (i, 0)),
        ],
        out_shape=[
            jax.ShapeDtypeStruct((BN, DIST_D), jnp.float32),
            jax.ShapeDtypeStruct((nblk, 1, BLK), jnp.int32),
            jax.ShapeDtypeStruct((BN, 128), jnp.float32),
            jax.ShapeDtypeStruct((BN, 128), jnp.float32),
        ],
    )(x2, We1, be1, We2, be2, cb8, Wd1a, Wd1b, bd1)

    emb = emb2.reshape(B, N, DIST_D)
    bin_idx = bins3.reshape(B, N)
    p1 = p1_2.reshape(B, N, 128)
    p2 = p2_2.reshape(B, N, 128)

    bins_split = jnp.argsort(bin_idx, axis=1).astype(jnp.int32)
    parts = jnp.take_along_axis(emb, bins_split[:, :, None], axis=1)
    parts = parts.reshape(B, n_bins, BIN_SIZE, DIST_D)

    valsT, idxT = pl.pallas_call(
        _stage3_body,
        grid=(B, n_bins),
        in_specs=[pl.BlockSpec((1, 1, BIN_SIZE, DIST_D), lambda b, c: (b, c, 0, 0))],
        out_specs=[
            pl.BlockSpec((1, 1, K, BIN_SIZE), lambda b, c: (b, c, 0, 0)),
            pl.BlockSpec((1, 1, K, BIN_SIZE), lambda b, c: (b, c, 0, 0)),
        ],
        out_shape=[
            jax.ShapeDtypeStruct((B, n_bins, K, BIN_SIZE), jnp.float32),
            jax.ShapeDtypeStruct((B, n_bins, K, BIN_SIZE), jnp.int32),
        ],
    )(parts)

    E = B * N * K
    boff = (jnp.arange(B, dtype=jnp.int32) * N)[:, None]
    edst3 = ((bins_split + boff)[:, :, None] * K +
             jnp.arange(K, dtype=jnp.int32)).reshape(B * n_bins, 50, 80)
    sc4 = pl.kernel(
        _sc4_body,
        mesh=plsc.VectorSubcoreMesh(core_axis_name="c", subcore_axis_name="s"),
        compiler_params=pltpu.CompilerParams(needs_layout_passes=False),
        out_type=[
            jax.ShapeDtypeStruct((E,), jnp.int32),
            jax.ShapeDtypeStruct((E,), jnp.float32),
            jax.ShapeDtypeStruct((E, 128), jnp.float32),
        ],
        scratch_types=[
            pltpu.VMEM((K * BIN_SIZE,), jnp.int32),
            pltpu.VMEM((K * BIN_SIZE,), jnp.float32),
            pltpu.VMEM((N,), jnp.int32),
            pltpu.VMEM((K * BIN_SIZE,), jnp.int32),
            pltpu.VMEM((K * BIN_SIZE,), jnp.float32),
            pltpu.VMEM((50, 80), jnp.int32),
            pltpu.VMEM((80, 128), jnp.float32),
            pltpu.SemaphoreType.DMA,
        ],
    )
    colsg, vals_e, g2 = sc4(idxT.reshape(B * n_bins, K * BIN_SIZE),
                            valsT.reshape(B * n_bins, K * BIN_SIZE),
                            bins_split, p2_2, edst3)
    cols_s = colsg.reshape(B, N, K) - boff[:, :, None]

    EB = BLK * K
    vals_flat = vals_e.reshape(nblk, 1, EB)
    edge_vals = pl.pallas_call(
        _stage5_body,
        grid=(nblk,),
        in_specs=[
            pl.BlockSpec((EB, 128), lambda i: (i, 0)),
            pl.BlockSpec((BLK, 128), lambda i: (i, 0)),
            pl.BlockSpec((1, 1, EB), lambda i: (i, 0, 0)),
            full((1, 128)), full((128, 1)),
            pl.BlockSpec((1, 1), lambda i: (0, 0)),
        ],
        out_specs=pl.BlockSpec((1, 1, EB), lambda i: (i, 0, 0)),
        out_shape=jax.ShapeDtypeStruct((nblk, 1, EB), jnp.float32),
    )(g2, p1_2, vals_flat, w512, Wd2, bd2.reshape(1, 1))
    edge_vals = edge_vals.reshape(E)

    batch_col = jnp.repeat(jnp.arange(B, dtype=jnp.int32), N * K)
    row_col = jnp.tile(jnp.repeat(jnp.arange(N, dtype=jnp.int32), K), B)
    indices = jnp.stack([batch_col, row_col, cols_s.reshape(E)], axis=1)
    return indices, edge_vals


# speedup vs baseline: 7.4373x; 1.0212x over previous
import functools

import jax
import jax.numpy as jnp
from jax import lax
from jax.experimental import pallas as pl
from jax.experimental.pallas import tpu as pltpu
from jax.experimental.pallas import tpu_sc as plsc

DIST_D = 128
BIN_SIZE = 500
K = 8
NBINS = 16


def _elu(x):
    return jnp.where(x > 0, x, jnp.exp(jnp.minimum(x, 0.0)) - 1.0)


def _stage1_body(x_ref, We1_ref, be1_ref, We2_ref, be2_ref, cb8_ref,
                 Wd1a_ref, Wd1b_ref, bd1_ref,
                 emb_ref, bin_ref, p1_ref, p2_ref):
    x = x_ref[...]
    h = _elu(x @ We1_ref[...] + be1_ref[...])
    emb = h @ We2_ref[...] + be2_ref[...]
    emb_ref[...] = emb
    mul = emb @ cb8_ref[...]
    cmul = jnp.concatenate([mul, -mul], axis=1)
    mx = jnp.max(cmul, axis=1, keepdims=True)
    iota = jax.lax.broadcasted_iota(jnp.int32, cmul.shape, 1)
    bin_ref[0, 0, :] = jnp.min(jnp.where(cmul == mx, iota, NBINS), axis=1)
    p1_ref[...] = x @ Wd1a_ref[...] + bd1_ref[...]
    p2_ref[...] = x @ Wd1b_ref[...]


def _stage3_body(parts_ref, vals_ref, idx_ref):
    pt = parts_ref[0, 0]
    s = jax.lax.dot_general(pt, pt, (((1,), (1,)), ((), ())))
    dm = jax.nn.sigmoid(s)
    iota = jax.lax.broadcasted_iota(jnp.int32, dm.shape, 1)
    for j in range(K):
        mx = jnp.max(dm, axis=1, keepdims=True)
        am = jnp.min(jnp.where(dm == mx, iota, BIN_SIZE), axis=1)
        vals_ref[0, 0, j, :] = mx[:, 0]
        idx_ref[0, 0, j, :] = am
        dm = jnp.where(iota == am[:, None], -jnp.inf, dm)


def _stage5_body(g_ref, p1_ref, vals_ref, w512_ref, Wd2_ref, bd2_ref, out_ref):
    rows = p1_ref.shape[0]
    g = g_ref[...].reshape(rows, K, DIST_D)
    z = g + p1_ref[...][:, None, :]
    z = z.reshape(rows * K, DIST_D)
    z = z + vals_ref[0, 0, :][:, None] * w512_ref[...]
    h = _elu(z)
    y = h @ Wd2_ref[...] + bd2_ref[0, 0]
    out_ref[0, 0, :] = jax.nn.sigmoid(y)[:, 0]


def _sc4_body(idxT, valsT, bins, p2, edst3,
              cols_out, valb_out, gout,
              idx_v, vals_v, bins_v, cols_v, valso_v, edst_v, g_v,
              sem, sem_s, sem_cv):
    wid = lax.axis_index("s") * 2 + lax.axis_index("c")
    b = wid // NBINS
    bn = wid % NBINS
    pw = bn * BIN_SIZE
    pltpu.sync_copy(idxT.at[wid], idx_v)
    pltpu.sync_copy(valsT.at[wid], vals_v)
    pltpu.sync_copy(bins.at[b], bins_v)
    pltpu.sync_copy(edst3.at[wid], edst_v)

    lane = lax.iota(jnp.int32, 16)
    j8 = lane & 7
    r2 = lane >> 3
    hi = r2 * 65536

    def body(t, carry):
        rowsel = 2 * t + r2
        loc = plsc.load_gather(idx_v, [j8 * BIN_SIZE + rowsel])
        val = plsc.load_gather(vals_v, [j8 * BIN_SIZE + rowsel])
        col = plsc.load_gather(bins_v, [pw + loc])
        ksort, vsort = plsc.sort_key_val(col + hi, val)
        cols_v[pl.ds(16 * t, 16)] = b * 8000 + (ksort - hi)
        valso_v[pl.ds(16 * t, 16)] = vsort
        return carry

    lax.fori_loop(0, BIN_SIZE // 2, body, 0)

    pending = []
    scat = {}
    NBUF = 4
    for c in range(50):
        sl = pl.ds(c * 80, 80)
        pending.append(pltpu.async_copy(cols_v.at[sl], cols_out.at[edst_v.at[c]], sem_cv))
        pending.append(pltpu.async_copy(valso_v.at[sl], valb_out.at[edst_v.at[c]], sem_cv))
        if c >= NBUF:
            scat[c - NBUF].wait()
        buf = g_v.at[c % NBUF]
        pltpu.async_copy(p2.at[cols_v.at[sl]], buf, sem).wait()
        scat[c] = pltpu.async_copy(buf, gout.at[edst_v.at[c]], sem_s)
    for d in pending + [scat[c] for c in range(50 - NBUF, 50)]:
        d.wait()


def kernel(inputs, We1, be1, We2, be2, Wd1, bd1, Wd2, bd2, cb):
    B, N, F = inputs.shape
    n_bins = N // BIN_SIZE
    BN = B * N
    BLK = 1000
    nblk = BN // BLK

    cb8 = cb[:, : n_bins // 2]
    Wd1a = Wd1[:F]
    Wd1b = Wd1[F:2 * F]
    w512 = Wd1[2 * F:2 * F + 1]

    x2 = inputs.reshape(BN, F)

    full = lambda shape: pl.BlockSpec(shape, lambda i: (0,) * len(shape))
    emb2, bins3, p1_2, p2_2 = pl.pallas_call(
        _stage1_body,
        grid=(nblk,),
        in_specs=[
            pl.BlockSpec((BLK, F), lambda i: (i, 0)),
            full((F, 128)), full((128,)), full((128, DIST_D)), full((DIST_D,)),
            full((DIST_D, n_bins // 2)),
            full((F, 128)), full((F, 128)), full((128,)),
        ],
        out_specs=[
            pl.BlockSpec((BLK, DIST_D), lambda i: (i, 0)),
            pl.BlockSpec((1, 1, BLK), lambda i: (i, 0, 0)),
            pl.BlockSpec((BLK, 128), lambda i: (i, 0)),
            pl.BlockSpec((BLK, 128), lambda i: (i, 0)),
        ],
        out_shape=[
            jax.ShapeDtypeStruct((BN, DIST_D), jnp.float32),
            jax.ShapeDtypeStruct((nblk, 1, BLK), jnp.int32),
            jax.ShapeDtypeStruct((BN, 128), jnp.float32),
            jax.ShapeDtypeStruct((BN, 128), jnp.float32),
        ],
    )(x2, We1, be1, We2, be2, cb8, Wd1a, Wd1b, bd1)

    emb = emb2.reshape(B, N, DIST_D)
    bin_idx = bins3.reshape(B, N)
    p1 = p1_2.reshape(B, N, 128)
    p2 = p2_2.reshape(B, N, 128)

    bins_split = jnp.argsort(bin_idx, axis=1).astype(jnp.int32)
    parts = jnp.take_along_axis(emb, bins_split[:, :, None], axis=1)
    parts = parts.reshape(B, n_bins, BIN_SIZE, DIST_D)

    valsT, idxT = pl.pallas_call(
        _stage3_body,
        grid=(B, n_bins),
        in_specs=[pl.BlockSpec((1, 1, BIN_SIZE, DIST_D), lambda b, c: (b, c, 0, 0))],
        out_specs=[
            pl.BlockSpec((1, 1, K, BIN_SIZE), lambda b, c: (b, c, 0, 0)),
            pl.BlockSpec((1, 1, K, BIN_SIZE), lambda b, c: (b, c, 0, 0)),
        ],
        out_shape=[
            jax.ShapeDtypeStruct((B, n_bins, K, BIN_SIZE), jnp.float32),
            jax.ShapeDtypeStruct((B, n_bins, K, BIN_SIZE), jnp.int32),
        ],
    )(parts)

    E = B * N * K
    boff = (jnp.arange(B, dtype=jnp.int32) * N)[:, None]
    edst3 = ((bins_split + boff)[:, :, None] * K +
             jnp.arange(K, dtype=jnp.int32)).reshape(B * n_bins, 50, 80)
    sc4 = pl.kernel(
        _sc4_body,
        mesh=plsc.VectorSubcoreMesh(core_axis_name="c", subcore_axis_name="s"),
        compiler_params=pltpu.CompilerParams(needs_layout_passes=False),
        out_type=[
            jax.ShapeDtypeStruct((E,), jnp.int32),
            jax.ShapeDtypeStruct((E,), jnp.float32),
            jax.ShapeDtypeStruct((E, 128), jnp.float32),
        ],
        scratch_types=[
            pltpu.VMEM((K * BIN_SIZE,), jnp.int32),
            pltpu.VMEM((K * BIN_SIZE,), jnp.float32),
            pltpu.VMEM((N,), jnp.int32),
            pltpu.VMEM((K * BIN_SIZE,), jnp.int32),
            pltpu.VMEM((K * BIN_SIZE,), jnp.float32),
            pltpu.VMEM((50, 80), jnp.int32),
            pltpu.VMEM((4, 80, 128), jnp.float32),
            pltpu.SemaphoreType.DMA,
            pltpu.SemaphoreType.DMA,
            pltpu.SemaphoreType.DMA,
        ],
    )
    colsg, vals_e, g2 = sc4(idxT.reshape(B * n_bins, K * BIN_SIZE),
                            valsT.reshape(B * n_bins, K * BIN_SIZE),
                            bins_split, p2_2, edst3)
    cols_s = colsg.reshape(B, N, K) - boff[:, :, None]

    EB = BLK * K
    vals_flat = vals_e.reshape(nblk, 1, EB)
    edge_vals = pl.pallas_call(
        _stage5_body,
        grid=(nblk,),
        in_specs=[
            pl.BlockSpec((EB, 128), lambda i: (i, 0)),
            pl.BlockSpec((BLK, 128), lambda i: (i, 0)),
            pl.BlockSpec((1, 1, EB), lambda i: (i, 0, 0)),
            full((1, 128)), full((128, 1)),
            pl.BlockSpec((1, 1), lambda i: (0, 0)),
        ],
        out_specs=pl.BlockSpec((1, 1, EB), lambda i: (i, 0, 0)),
        out_shape=jax.ShapeDtypeStruct((nblk, 1, EB), jnp.float32),
    )(g2, p1_2, vals_flat, w512, Wd2, bd2.reshape(1, 1))
    edge_vals = edge_vals.reshape(E)

    batch_col = jnp.repeat(jnp.arange(B, dtype=jnp.int32), N * K)
    row_col = jnp.tile(jnp.repeat(jnp.arange(N, dtype=jnp.int32), K), B)
    indices = jnp.stack([batch_col, row_col, cols_s.reshape(E)], axis=1)
    return indices, edge_vals


# speedup vs baseline: 13.0166x; 1.7502x over previous
import functools

import jax
import jax.numpy as jnp
from jax import lax
from jax.experimental import pallas as pl
from jax.experimental.pallas import tpu as pltpu
from jax.experimental.pallas import tpu_sc as plsc

DIST_D = 128
BIN_SIZE = 500
K = 8
NBINS = 16


def _elu(x):
    return jnp.where(x > 0, x, jnp.exp(jnp.minimum(x, 0.0)) - 1.0)


def _stage1_body(x_ref, We1_ref, be1_ref, We2_ref, be2_ref, cb8_ref,
                 Wd1a_ref, Wd1b_ref, bd1_ref,
                 emb_ref, bin_ref, p1_ref, p2_ref):
    x = x_ref[...]
    h = _elu(x @ We1_ref[...] + be1_ref[...])
    emb = h @ We2_ref[...] + be2_ref[...]
    emb_ref[...] = emb
    mul = emb @ cb8_ref[...]
    cmul = jnp.concatenate([mul, -mul], axis=1)
    mx = jnp.max(cmul, axis=1, keepdims=True)
    iota = jax.lax.broadcasted_iota(jnp.int32, cmul.shape, 1)
    bin_ref[0, 0, :] = jnp.min(jnp.where(cmul == mx, iota, NBINS), axis=1)
    p1_ref[...] = x @ Wd1a_ref[...] + bd1_ref[...]
    p2_ref[...] = x @ Wd1b_ref[...]


def _stage3_body(parts_ref, vals_ref, idx_ref):
    pt = parts_ref[0, 0]
    s = jax.lax.dot_general(pt, pt, (((1,), (1,)), ((), ())))
    dm = jax.nn.sigmoid(s)
    iota = jax.lax.broadcasted_iota(jnp.int32, dm.shape, 1)
    for j in range(K):
        mx = jnp.max(dm, axis=1, keepdims=True)
        am = jnp.min(jnp.where(dm == mx, iota, BIN_SIZE), axis=1)
        vals_ref[0, 0, j, :] = mx[:, 0]
        idx_ref[0, 0, j, :] = am
        dm = jnp.where(iota == am[:, None], -jnp.inf, dm)


def _stage5_body(g_ref, p1_ref, vals_ref, w512_ref, Wd2_ref, bd2_ref, out_ref):
    rows = p1_ref.shape[0]
    g = g_ref[...].reshape(rows, K, DIST_D)
    z = g + p1_ref[...][:, None, :]
    z = z.reshape(rows * K, DIST_D)
    z = z + vals_ref[0, 0, :][:, None] * w512_ref[...]
    h = _elu(z)
    y = h @ Wd2_ref[...] + bd2_ref[0, 0]
    out_ref[0, 0, :] = jax.nn.sigmoid(y)[:, 0]


def _sc4_body(idxT, valsT, bins, p2, edst3, sidx3,
              packed_out, gout,
              idx_v, vals_v, bins_v, cols_v, packed_v, edst_v, sidx_v, g_v,
              sem, sem_s, sem_cv):
    wid = lax.axis_index("s") * 2 + lax.axis_index("c")
    b = wid // NBINS
    bn = wid % NBINS
    pw = bn * BIN_SIZE
    pltpu.sync_copy(idxT.at[wid], idx_v)
    pltpu.sync_copy(valsT.at[wid], vals_v)
    pltpu.sync_copy(bins.at[b], bins_v)
    pltpu.sync_copy(edst3.at[wid], edst_v)
    pltpu.sync_copy(sidx3.at[wid], sidx_v)

    lane = lax.iota(jnp.int32, 16)
    j8 = lane & 7
    r2 = lane >> 3
    hi = r2 * 65536

    def body(t, carry):
        rowsel = 2 * t + r2
        loc = plsc.load_gather(idx_v, [j8 * BIN_SIZE + rowsel])
        val = plsc.load_gather(vals_v, [j8 * BIN_SIZE + rowsel])
        col = plsc.load_gather(bins_v, [pw + loc])
        ksort, vsort = plsc.sort_key_val(col + hi, val)
        cols_sorted = ksort - hi
        cols_v[pl.ds(16 * t, 16)] = b * 8000 + cols_sorted
        plsc.store_scatter(packed_v, [rowsel, j8], cols_sorted)
        plsc.store_scatter(packed_v, [rowsel, j8 + 8],
                           plsc.bitcast(vsort, jnp.int32))
        return carry

    lax.fori_loop(0, BIN_SIZE // 2, body, 0)

    pending = []
    for c in range(4):
        pending.append(pltpu.async_copy(packed_v.at[pl.ds(c * 125, 125)],
                                        packed_out.at[sidx_v.at[c]], sem_cv))
    scat = {}
    NBUF = 4
    for c in range(50):
        sl = pl.ds(c * 80, 80)
        if c >= NBUF:
            scat[c - NBUF].wait()
        buf = g_v.at[c % NBUF]
        pltpu.async_copy(p2.at[cols_v.at[sl]], buf, sem).wait()
        scat[c] = pltpu.async_copy(buf, gout.at[edst_v.at[c]], sem_s)
    for d in pending + [scat[c] for c in range(50 - NBUF, 50)]:
        d.wait()


def kernel(inputs, We1, be1, We2, be2, Wd1, bd1, Wd2, bd2, cb):
    B, N, F = inputs.shape
    n_bins = N // BIN_SIZE
    BN = B * N
    BLK = 1000
    nblk = BN // BLK

    cb8 = cb[:, : n_bins // 2]
    Wd1a = Wd1[:F]
    Wd1b = Wd1[F:2 * F]
    w512 = Wd1[2 * F:2 * F + 1]

    x2 = inputs.reshape(BN, F)

    full = lambda shape: pl.BlockSpec(shape, lambda i: (0,) * len(shape))
    emb2, bins3, p1_2, p2_2 = pl.pallas_call(
        _stage1_body,
        grid=(nblk,),
        in_specs=[
            pl.BlockSpec((BLK, F), lambda i: (i, 0)),
            full((F, 128)), full((128,)), full((128, DIST_D)), full((DIST_D,)),
            full((DIST_D, n_bins // 2)),
            full((F, 128)), full((F, 128)), full((128,)),
        ],
        out_specs=[
            pl.BlockSpec((BLK, DIST_D), lambda i: (i, 0)),
            pl.BlockSpec((1, 1, BLK), lambda i: (i, 0, 0)),
            pl.BlockSpec((BLK, 128), lambda i: (i, 0)),
            pl.BlockSpec((BLK, 128), lambda i: (i, 0)),
        ],
        out_shape=[
            jax.ShapeDtypeStruct((BN, DIST_D), jnp.float32),
            jax.ShapeDtypeStruct((nblk, 1, BLK), jnp.int32),
            jax.ShapeDtypeStruct((BN, 128), jnp.float32),
            jax.ShapeDtypeStruct((BN, 128), jnp.float32),
        ],
    )(x2, We1, be1, We2, be2, cb8, Wd1a, Wd1b, bd1)

    emb = emb2.reshape(B, N, DIST_D)
    bin_idx = bins3.reshape(B, N)
    p1 = p1_2.reshape(B, N, 128)
    p2 = p2_2.reshape(B, N, 128)

    bins_split = jnp.argsort(bin_idx, axis=1).astype(jnp.int32)
    parts = jnp.take_along_axis(emb, bins_split[:, :, None], axis=1)
    parts = parts.reshape(B, n_bins, BIN_SIZE, DIST_D)

    valsT, idxT = pl.pallas_call(
        _stage3_body,
        grid=(B, n_bins),
        in_specs=[pl.BlockSpec((1, 1, BIN_SIZE, DIST_D), lambda b, c: (b, c, 0, 0))],
        out_specs=[
            pl.BlockSpec((1, 1, K, BIN_SIZE), lambda b, c: (b, c, 0, 0)),
            pl.BlockSpec((1, 1, K, BIN_SIZE), lambda b, c: (b, c, 0, 0)),
        ],
        out_shape=[
            jax.ShapeDtypeStruct((B, n_bins, K, BIN_SIZE), jnp.float32),
            jax.ShapeDtypeStruct((B, n_bins, K, BIN_SIZE), jnp.int32),
        ],
    )(parts)

    E = B * N * K
    boff = (jnp.arange(B, dtype=jnp.int32) * N)[:, None]
    edst3 = ((bins_split + boff)[:, :, None] * K +
             jnp.arange(K, dtype=jnp.int32)).reshape(B * n_bins, 50, 80)
    sidx3 = (bins_split + boff).reshape(B * n_bins, 4, 125)
    sc4 = pl.kernel(
        _sc4_body,
        mesh=plsc.VectorSubcoreMesh(core_axis_name="c", subcore_axis_name="s"),
        compiler_params=pltpu.CompilerParams(needs_layout_passes=False, use_tc_tiling_on_sc=False),
        out_type=[
            jax.ShapeDtypeStruct((BN, 2 * K), jnp.int32),
            jax.ShapeDtypeStruct((E, 128), jnp.float32),
        ],
        scratch_types=[
            pltpu.VMEM((K * BIN_SIZE,), jnp.int32),
            pltpu.VMEM((K * BIN_SIZE,), jnp.float32),
            pltpu.VMEM((N,), jnp.int32),
            pltpu.VMEM((K * BIN_SIZE,), jnp.int32),
            pltpu.VMEM((BIN_SIZE, 2 * K), jnp.int32),
            pltpu.VMEM((50, 80), jnp.int32),
            pltpu.VMEM((4, 125), jnp.int32),
            pltpu.VMEM((4, 80, 128), jnp.float32),
            pltpu.SemaphoreType.DMA,
            pltpu.SemaphoreType.DMA,
            pltpu.SemaphoreType.DMA,
        ],
    )
    packed, g2 = sc4(idxT.reshape(B * n_bins, K * BIN_SIZE),
                     valsT.reshape(B * n_bins, K * BIN_SIZE),
                     bins_split, p2_2, edst3, sidx3)
    cols_s = packed[:, :K].reshape(B, N, K)
    vals_e = jax.lax.bitcast_convert_type(packed[:, K:], jnp.float32).reshape(E)

    EB = BLK * K
    vals_flat = vals_e.reshape(nblk, 1, EB)
    edge_vals = pl.pallas_call(
        _stage5_body,
        grid=(nblk,),
        in_specs=[
            pl.BlockSpec((EB, 128), lambda i: (i, 0)),
            pl.BlockSpec((BLK, 128), lambda i: (i, 0)),
            pl.BlockSpec((1, 1, EB), lambda i: (i, 0, 0)),
            full((1, 128)), full((128, 1)),
            pl.BlockSpec((1, 1), lambda i: (0, 0)),
        ],
        out_specs=pl.BlockSpec((1, 1, EB), lambda i: (i, 0, 0)),
        out_shape=jax.ShapeDtypeStruct((nblk, 1, EB), jnp.float32),
    )(g2, p1_2, vals_flat, w512, Wd2, bd2.reshape(1, 1))
    edge_vals = edge_vals.reshape(E)

    batch_col = jnp.repeat(jnp.arange(B, dtype=jnp.int32), N * K)
    row_col = jnp.tile(jnp.repeat(jnp.arange(N, dtype=jnp.int32), K), B)
    indices = jnp.stack([batch_col, row_col, cols_s.reshape(E)], axis=1)
    return indices, edge_vals


# speedup vs baseline: 13.2533x; 1.0182x over previous
import functools

import jax
import jax.numpy as jnp
from jax import lax
from jax.experimental import pallas as pl
from jax.experimental.pallas import tpu as pltpu
from jax.experimental.pallas import tpu_sc as plsc

DIST_D = 128
BIN_SIZE = 500
K = 8
NBINS = 16


def _elu(x):
    return jnp.where(x > 0, x, jnp.exp(jnp.minimum(x, 0.0)) - 1.0)


def _stage1_body(x_ref, We1_ref, be1_ref, We2_ref, be2_ref, cb8_ref,
                 Wd1a_ref, Wd1b_ref, bd1_ref,
                 emb_ref, bin_ref, p1_ref, p2_ref):
    x = x_ref[...]
    h = _elu(x @ We1_ref[...] + be1_ref[...])
    emb = h @ We2_ref[...] + be2_ref[...]
    emb_ref[...] = emb
    mul = emb @ cb8_ref[...]
    cmul = jnp.concatenate([mul, -mul], axis=1)
    mx = jnp.max(cmul, axis=1, keepdims=True)
    iota = jax.lax.broadcasted_iota(jnp.int32, cmul.shape, 1)
    bin_ref[0, 0, :] = jnp.min(jnp.where(cmul == mx, iota, NBINS), axis=1)
    p1_ref[...] = x @ Wd1a_ref[...] + bd1_ref[...]
    p2_ref[...] = x @ Wd1b_ref[...]


def _stage3_body(parts_ref, vals_ref, idx_ref):
    pt = parts_ref[0, 0]
    dm = jax.lax.dot_general(pt, pt, (((1,), (1,)), ((), ())))
    iota = jax.lax.broadcasted_iota(jnp.int32, dm.shape, 1)
    for j in range(K):
        mx = jnp.max(dm, axis=1, keepdims=True)
        am = jnp.min(jnp.where(dm == mx, iota, BIN_SIZE), axis=1)
        vals_ref[0, 0, j, :] = jax.nn.sigmoid(mx[:, 0])
        idx_ref[0, 0, j, :] = am
        dm = jnp.where(iota == am[:, None], -jnp.inf, dm)


def _stage5_body(g_ref, p1_ref, vals_ref, w512_ref, Wd2_ref, bd2_ref, out_ref):
    rows = p1_ref.shape[0]
    g = g_ref[...].reshape(rows, K, DIST_D)
    z = g + p1_ref[...][:, None, :]
    z = z.reshape(rows * K, DIST_D)
    z = z + vals_ref[0, 0, :][:, None] * w512_ref[...]
    h = _elu(z)
    y = h @ Wd2_ref[...] + bd2_ref[0, 0]
    out_ref[0, 0, :] = jax.nn.sigmoid(y)[:, 0]


def _sc4_body(idxT, valsT, bins, p2, edst3, sidx3,
              packed_out, gout,
              idx_v, vals_v, bins_v, cols_v, packed_v, edst_v, sidx_v, g_v,
              sem, sem_s, sem_cv):
    wid = lax.axis_index("s") * 2 + lax.axis_index("c")
    b = wid // NBINS
    bn = wid % NBINS
    pw = bn * BIN_SIZE
    pltpu.sync_copy(idxT.at[wid], idx_v)
    pltpu.sync_copy(valsT.at[wid], vals_v)
    pltpu.sync_copy(bins.at[b], bins_v)
    pltpu.sync_copy(edst3.at[wid], edst_v)
    pltpu.sync_copy(sidx3.at[wid], sidx_v)

    lane = lax.iota(jnp.int32, 16)
    j8 = lane & 7
    r2 = lane >> 3
    hi = r2 * 65536

    def body(t, carry):
        rowsel = 2 * t + r2
        loc = plsc.load_gather(idx_v, [j8 * BIN_SIZE + rowsel])
        val = plsc.load_gather(vals_v, [j8 * BIN_SIZE + rowsel])
        col = plsc.load_gather(bins_v, [pw + loc])
        ksort, vsort = plsc.sort_key_val(col + hi, val)
        cols_sorted = ksort - hi
        cols_v[pl.ds(16 * t, 16)] = b * 8000 + cols_sorted
        plsc.store_scatter(packed_v, [rowsel, j8], cols_sorted)
        plsc.store_scatter(packed_v, [rowsel, j8 + 8],
                           plsc.bitcast(vsort, jnp.int32))
        return carry

    lax.fori_loop(0, BIN_SIZE // 2, body, 0)

    pending = []
    for c in range(4):
        pending.append(pltpu.async_copy(packed_v.at[pl.ds(c * 125, 125)],
                                        packed_out.at[sidx_v.at[c]], sem_cv))
    scat = {}
    NBUF = 4
    for c in range(50):
        sl = pl.ds(c * 80, 80)
        if c >= NBUF:
            scat[c - NBUF].wait()
        buf = g_v.at[c % NBUF]
        pltpu.async_copy(p2.at[cols_v.at[sl]], buf, sem).wait()
        scat[c] = pltpu.async_copy(buf, gout.at[edst_v.at[c]], sem_s)
    for d in pending + [scat[c] for c in range(50 - NBUF, 50)]:
        d.wait()


def kernel(inputs, We1, be1, We2, be2, Wd1, bd1, Wd2, bd2, cb):
    B, N, F = inputs.shape
    n_bins = N // BIN_SIZE
    BN = B * N
    BLK = 1000
    nblk = BN // BLK

    cb8 = cb[:, : n_bins // 2]
    Wd1a = Wd1[:F]
    Wd1b = Wd1[F:2 * F]
    w512 = Wd1[2 * F:2 * F + 1]

    x2 = inputs.reshape(BN, F)

    full = lambda shape: pl.BlockSpec(shape, lambda i: (0,) * len(shape))
    emb2, bins3, p1_2, p2_2 = pl.pallas_call(
        _stage1_body,
        grid=(nblk,),
        in_specs=[
            pl.BlockSpec((BLK, F), lambda i: (i, 0)),
            full((F, 128)), full((128,)), full((128, DIST_D)), full((DIST_D,)),
            full((DIST_D, n_bins // 2)),
            full((F, 128)), full((F, 128)), full((128,)),
        ],
        out_specs=[
            pl.BlockSpec((BLK, DIST_D), lambda i: (i, 0)),
            pl.BlockSpec((1, 1, BLK), lambda i: (i, 0, 0)),
            pl.BlockSpec((BLK, 128), lambda i: (i, 0)),
            pl.BlockSpec((BLK, 128), lambda i: (i, 0)),
        ],
        out_shape=[
            jax.ShapeDtypeStruct((BN, DIST_D), jnp.float32),
            jax.ShapeDtypeStruct((nblk, 1, BLK), jnp.int32),
            jax.ShapeDtypeStruct((BN, 128), jnp.float32),
            jax.ShapeDtypeStruct((BN, 128), jnp.float32),
        ],
    )(x2, We1, be1, We2, be2, cb8, Wd1a, Wd1b, bd1)

    emb = emb2.reshape(B, N, DIST_D)
    bin_idx = bins3.reshape(B, N)
    p1 = p1_2.reshape(B, N, 128)
    p2 = p2_2.reshape(B, N, 128)

    bins_split = jnp.argsort(bin_idx, axis=1).astype(jnp.int32)
    parts = jnp.take_along_axis(emb, bins_split[:, :, None], axis=1)
    parts = parts.reshape(B, n_bins, BIN_SIZE, DIST_D)

    valsT, idxT = pl.pallas_call(
        _stage3_body,
        grid=(B, n_bins),
        in_specs=[pl.BlockSpec((1, 1, BIN_SIZE, DIST_D), lambda b, c: (b, c, 0, 0))],
        out_specs=[
            pl.BlockSpec((1, 1, K, BIN_SIZE), lambda b, c: (b, c, 0, 0)),
            pl.BlockSpec((1, 1, K, BIN_SIZE), lambda b, c: (b, c, 0, 0)),
        ],
        out_shape=[
            jax.ShapeDtypeStruct((B, n_bins, K, BIN_SIZE), jnp.float32),
            jax.ShapeDtypeStruct((B, n_bins, K, BIN_SIZE), jnp.int32),
        ],
    )(parts)

    E = B * N * K
    boff = (jnp.arange(B, dtype=jnp.int32) * N)[:, None]
    edst3 = ((bins_split + boff)[:, :, None] * K +
             jnp.arange(K, dtype=jnp.int32)).reshape(B * n_bins, 50, 80)
    sidx3 = (bins_split + boff).reshape(B * n_bins, 4, 125)
    sc4 = pl.kernel(
        _sc4_body,
        mesh=plsc.VectorSubcoreMesh(core_axis_name="c", subcore_axis_name="s"),
        compiler_params=pltpu.CompilerParams(needs_layout_passes=False, use_tc_tiling_on_sc=False),
        out_type=[
            jax.ShapeDtypeStruct((BN, 2 * K), jnp.int32),
            jax.ShapeDtypeStruct((E, 128), jnp.float32),
        ],
        scratch_types=[
            pltpu.VMEM((K * BIN_SIZE,), jnp.int32),
            pltpu.VMEM((K * BIN_SIZE,), jnp.float32),
            pltpu.VMEM((N,), jnp.int32),
            pltpu.VMEM((K * BIN_SIZE,), jnp.int32),
            pltpu.VMEM((BIN_SIZE, 2 * K), jnp.int32),
            pltpu.VMEM((50, 80), jnp.int32),
            pltpu.VMEM((4, 125), jnp.int32),
            pltpu.VMEM((4, 80, 128), jnp.float32),
            pltpu.SemaphoreType.DMA,
            pltpu.SemaphoreType.DMA,
            pltpu.SemaphoreType.DMA,
        ],
    )
    packed, g2 = sc4(idxT.reshape(B * n_bins, K * BIN_SIZE),
                     valsT.reshape(B * n_bins, K * BIN_SIZE),
                     bins_split, p2_2, edst3, sidx3)
    cols_s = packed[:, :K].reshape(B, N, K)
    vals_e = jax.lax.bitcast_convert_type(packed[:, K:], jnp.float32).reshape(E)

    EB = BLK * K
    vals_flat = vals_e.reshape(nblk, 1, EB)
    edge_vals = pl.pallas_call(
        _stage5_body,
        grid=(nblk,),
        in_specs=[
            pl.BlockSpec((EB, 128), lambda i: (i, 0)),
            pl.BlockSpec((BLK, 128), lambda i: (i, 0)),
            pl.BlockSpec((1, 1, EB), lambda i: (i, 0, 0)),
            full((1, 128)), full((128, 1)),
            pl.BlockSpec((1, 1), lambda i: (0, 0)),
        ],
        out_specs=pl.BlockSpec((1, 1, EB), lambda i: (i, 0, 0)),
        out_shape=jax.ShapeDtypeStruct((nblk, 1, EB), jnp.float32),
    )(g2, p1_2, vals_flat, w512, Wd2, bd2.reshape(1, 1))
    edge_vals = edge_vals.reshape(E)

    batch_col = jnp.repeat(jnp.arange(B, dtype=jnp.int32), N * K)
    row_col = jnp.tile(jnp.repeat(jnp.arange(N, dtype=jnp.int32), K), B)
    indices = jnp.stack([batch_col, row_col, cols_s.reshape(E)], axis=1)
    return indices, edge_vals
